# Initial kernel scaffold; baseline (speedup 1.0000x reference)
import jax
import jax.numpy as jnp
from jax.experimental import pallas as pl


def kernel(h, W1, b1, W2, b2, epoch, top_k):
    raise NotImplementedError("write your pallas kernel here")


# speedup vs baseline: 4.5648x; 4.5648x over previous
import functools

import jax
import jax.numpy as jnp
from jax import lax
from jax.experimental import pallas as pl

_HIDDEN = 768
_EXPERTS = 64
_BT = 1024


def _gating_body(flag_ref, h_ref, w1_ref, b1_ref, w2_ref, b2_ref, out_ref):
    a1 = jnp.tanh(
        jnp.dot(h_ref[...], w1_ref[...], preferred_element_type=jnp.float32)
        + b1_ref[...]
    )
    logits = (
        jnp.dot(a1, w2_ref[...], preferred_element_type=jnp.float32) + b2_ref[...]
    )

    m1 = jnp.max(logits, axis=-1, keepdims=True)
    ids = lax.broadcasted_iota(jnp.int32, logits.shape, 1)
    idx1 = jnp.min(
        jnp.where(logits == m1, ids, _EXPERTS), axis=-1, keepdims=True
    )
    m2 = jnp.max(jnp.where(ids == idx1, -jnp.inf, logits), axis=-1, keepdims=True)

    e = jnp.exp(logits - m1)
    dense = e / jnp.sum(e, axis=-1, keepdims=True)
    keep = logits >= m2
    se = jnp.where(keep, e, 0.0)
    sparse = se / jnp.sum(se, axis=-1, keepdims=True)

    use_dense = flag_ref[0, 0] != 0
    out_ref[...] = jnp.where(use_dense, dense, sparse)


@jax.jit
def _gating(h, W1, b1, W2, b2, flag):
    tokens = h.shape[0]
    grid = (tokens // _BT,)
    return pl.pallas_call(
        _gating_body,
        grid=grid,
        in_specs=[
            pl.BlockSpec((1, 1), lambda i: (0, 0)),
            pl.BlockSpec((_BT, _HIDDEN), lambda i: (i, 0)),
            pl.BlockSpec((_HIDDEN, _HIDDEN), lambda i: (0, 0)),
            pl.BlockSpec((1, _HIDDEN), lambda i: (0, 0)),
            pl.BlockSpec((_HIDDEN, _EXPERTS), lambda i: (0, 0)),
            pl.BlockSpec((1, _EXPERTS), lambda i: (0, 0)),
        ],
        out_specs=pl.BlockSpec((_BT, _EXPERTS), lambda i: (i, 0)),
        out_shape=jax.ShapeDtypeStruct((tokens, _EXPERTS), jnp.float32),
    )(flag, h, W1.reshape(_HIDDEN, _HIDDEN), b1.reshape(1, _HIDDEN),
      W2.reshape(_HIDDEN, _EXPERTS), b2.reshape(1, _EXPERTS))


def kernel(h, W1, b1, W2, b2, epoch, top_k):
    warmup_epochs = 0
    if epoch is None or top_k is None:
        flag = jnp.ones((1, 1), jnp.float32)
    else:
        use_dense = (epoch < warmup_epochs) | (top_k <= 0)
        flag = jnp.asarray(use_dense, jnp.float32).reshape(1, 1)
    return _gating(h, W1, b1, W2, b2, flag)


# speedup vs baseline: 4.6556x; 1.0199x over previous
import functools

import jax
import jax.numpy as jnp
from jax import lax
from jax.experimental import pallas as pl

_HIDDEN = 768
_EXPERTS = 64
_BT = 1024


def _gating_body(flag_ref, h_ref, w1_ref, b1_ref, w2_ref, b2_ref, out_ref):
    a1 = jnp.tanh(
        jnp.dot(h_ref[...], w1_ref[...], preferred_element_type=jnp.float32)
        + b1_ref[...]
    )
    logits = (
        jnp.dot(a1, w2_ref[...], preferred_element_type=jnp.float32) + b2_ref[...]
    )

    m1 = jnp.max(logits, axis=-1, keepdims=True)
    ids = lax.broadcasted_iota(jnp.int32, logits.shape, 1)
    idx1 = jnp.min(
        jnp.where(logits == m1, ids, _EXPERTS), axis=-1, keepdims=True
    )
    m2 = jnp.max(jnp.where(ids == idx1, -jnp.inf, logits), axis=-1, keepdims=True)

    use_dense = flag_ref[0, 0] != 0

    @pl.when(jnp.logical_not(use_dense))
    def _sparse():
        t = jnp.exp(m2 - m1)
        p2 = t / (1.0 + t)
        p1 = 1.0 - p2
        out_ref[...] = jnp.where(
            ids == idx1, p1, jnp.where(logits >= m2, p2, 0.0)
        )

    @pl.when(use_dense)
    def _dense():
        e = jnp.exp(logits - m1)
        out_ref[...] = e / jnp.sum(e, axis=-1, keepdims=True)


@jax.jit
def _gating(h, W1, b1, W2, b2, flag):
    tokens = h.shape[0]
    grid = (tokens // _BT,)
    return pl.pallas_call(
        _gating_body,
        grid=grid,
        in_specs=[
            pl.BlockSpec((1, 1), lambda i: (0, 0)),
            pl.BlockSpec((_BT, _HIDDEN), lambda i: (i, 0)),
            pl.BlockSpec((_HIDDEN, _HIDDEN), lambda i: (0, 0)),
            pl.BlockSpec((1, _HIDDEN), lambda i: (0, 0)),
            pl.BlockSpec((_HIDDEN, _EXPERTS), lambda i: (0, 0)),
            pl.BlockSpec((1, _EXPERTS), lambda i: (0, 0)),
        ],
        out_specs=pl.BlockSpec((_BT, _EXPERTS), lambda i: (i, 0)),
        out_shape=jax.ShapeDtypeStruct((tokens, _EXPERTS), jnp.float32),
    )(flag, h, W1.reshape(_HIDDEN, _HIDDEN), b1.reshape(1, _HIDDEN),
      W2.reshape(_HIDDEN, _EXPERTS), b2.reshape(1, _EXPERTS))


def kernel(h, W1, b1, W2, b2, epoch, top_k):
    warmup_epochs = 0
    if epoch is None or top_k is None:
        flag = jnp.ones((1, 1), jnp.float32)
    else:
        use_dense = (epoch < warmup_epochs) | (top_k <= 0)
        flag = jnp.asarray(use_dense, jnp.float32).reshape(1, 1)
    return _gating(h, W1, b1, W2, b2, flag)


# speedup vs baseline: 4.9991x; 1.0738x over previous
import functools

import jax
import jax.numpy as jnp
from jax import lax
from jax.experimental import pallas as pl

_HIDDEN = 768
_EXPERTS = 64
_BT = 2048


def _gating_body(flag_ref, h_ref, w1_ref, b1_ref, w2_ref, b2_ref, out_ref):
    a1 = jnp.tanh(
        jnp.dot(h_ref[...], w1_ref[...], preferred_element_type=jnp.float32)
        + b1_ref[...]
    )
    logits = (
        jnp.dot(a1, w2_ref[...], preferred_element_type=jnp.float32) + b2_ref[...]
    )

    m1 = jnp.max(logits, axis=-1, keepdims=True)
    ids = lax.broadcasted_iota(jnp.int32, logits.shape, 1)
    idx1 = jnp.min(
        jnp.where(logits == m1, ids, _EXPERTS), axis=-1, keepdims=True
    )
    m2 = jnp.max(jnp.where(ids == idx1, -jnp.inf, logits), axis=-1, keepdims=True)

    use_dense = flag_ref[0, 0] != 0

    @pl.when(jnp.logical_not(use_dense))
    def _sparse():
        t = jnp.exp(m2 - m1)
        p2 = t / (1.0 + t)
        p1 = 1.0 - p2
        out_ref[...] = jnp.where(
            ids == idx1, p1, jnp.where(logits >= m2, p2, 0.0)
        )

    @pl.when(use_dense)
    def _dense():
        e = jnp.exp(logits - m1)
        out_ref[...] = e / jnp.sum(e, axis=-1, keepdims=True)


@jax.jit
def _gating(h, W1, b1, W2, b2, flag):
    tokens = h.shape[0]
    grid = (tokens // _BT,)
    return pl.pallas_call(
        _gating_body,
        grid=grid,
        in_specs=[
            pl.BlockSpec((1, 1), lambda i: (0, 0)),
            pl.BlockSpec((_BT, _HIDDEN), lambda i: (i, 0)),
            pl.BlockSpec((_HIDDEN, _HIDDEN), lambda i: (0, 0)),
            pl.BlockSpec((1, _HIDDEN), lambda i: (0, 0)),
            pl.BlockSpec((_HIDDEN, _EXPERTS), lambda i: (0, 0)),
            pl.BlockSpec((1, _EXPERTS), lambda i: (0, 0)),
        ],
        out_specs=pl.BlockSpec((_BT, _EXPERTS), lambda i: (i, 0)),
        out_shape=jax.ShapeDtypeStruct((tokens, _EXPERTS), jnp.float32),
    )(flag, h, W1.reshape(_HIDDEN, _HIDDEN), b1.reshape(1, _HIDDEN),
      W2.reshape(_HIDDEN, _EXPERTS), b2.reshape(1, _EXPERTS))


def kernel(h, W1, b1, W2, b2, epoch, top_k):
    warmup_epochs = 0
    if epoch is None or top_k is None:
        flag = jnp.ones((1, 1), jnp.float32)
    else:
        use_dense = (epoch < warmup_epochs) | (top_k <= 0)
        flag = jnp.asarray(use_dense, jnp.float32).reshape(1, 1)
    return _gating(h, W1, b1, W2, b2, flag)


# speedup vs baseline: 5.1608x; 1.0323x over previous
import functools

import jax
import jax.numpy as jnp
from jax import lax
from jax.experimental import pallas as pl

_HIDDEN = 768
_EXPERTS = 64
_BT = 4096


def _gating_body(flag_ref, h_ref, w1_ref, b1_ref, w2_ref, b2_ref, out_ref):
    a1 = jnp.tanh(
        jnp.dot(h_ref[...], w1_ref[...], preferred_element_type=jnp.float32)
        + b1_ref[...]
    )
    logits = (
        jnp.dot(a1, w2_ref[...], preferred_element_type=jnp.float32) + b2_ref[...]
    )

    m1 = jnp.max(logits, axis=-1, keepdims=True)
    ids = lax.broadcasted_iota(jnp.int32, logits.shape, 1)
    idx1 = jnp.min(
        jnp.where(logits == m1, ids, _EXPERTS), axis=-1, keepdims=True
    )
    m2 = jnp.max(jnp.where(ids == idx1, -jnp.inf, logits), axis=-1, keepdims=True)

    use_dense = flag_ref[0, 0] != 0

    @pl.when(jnp.logical_not(use_dense))
    def _sparse():
        t = jnp.exp(m2 - m1)
        p2 = t / (1.0 + t)
        p1 = 1.0 - p2
        out_ref[...] = jnp.where(
            ids == idx1, p1, jnp.where(logits >= m2, p2, 0.0)
        )

    @pl.when(use_dense)
    def _dense():
        e = jnp.exp(logits - m1)
        out_ref[...] = e / jnp.sum(e, axis=-1, keepdims=True)


@jax.jit
def _gating(h, W1, b1, W2, b2, flag):
    tokens = h.shape[0]
    grid = (tokens // _BT,)
    return pl.pallas_call(
        _gating_body,
        grid=grid,
        in_specs=[
            pl.BlockSpec((1, 1), lambda i: (0, 0)),
            pl.BlockSpec((_BT, _HIDDEN), lambda i: (i, 0)),
            pl.BlockSpec((_HIDDEN, _HIDDEN), lambda i: (0, 0)),
            pl.BlockSpec((1, _HIDDEN), lambda i: (0, 0)),
            pl.BlockSpec((_HIDDEN, _EXPERTS), lambda i: (0, 0)),
            pl.BlockSpec((1, _EXPERTS), lambda i: (0, 0)),
        ],
        out_specs=pl.BlockSpec((_BT, _EXPERTS), lambda i: (i, 0)),
        out_shape=jax.ShapeDtypeStruct((tokens, _EXPERTS), jnp.float32),
    )(flag, h, W1.reshape(_HIDDEN, _HIDDEN), b1.reshape(1, _HIDDEN),
      W2.reshape(_HIDDEN, _EXPERTS), b2.reshape(1, _EXPERTS))


def kernel(h, W1, b1, W2, b2, epoch, top_k):
    warmup_epochs = 0
    if epoch is None or top_k is None:
        flag = jnp.ones((1, 1), jnp.float32)
    else:
        use_dense = (epoch < warmup_epochs) | (top_k <= 0)
        flag = jnp.asarray(use_dense, jnp.float32).reshape(1, 1)
    return _gating(h, W1, b1, W2, b2, flag)


# speedup vs baseline: 5.6732x; 1.0993x over previous
import functools

import jax
import jax.numpy as jnp
from jax import lax
from jax.experimental import pallas as pl

_HIDDEN = 768
_EXPERTS = 64
_BT = 4096


def _gating_body(flag_ref, h_ref, w1_ref, b1_ref, w2_ref, b2_ref, out_ref):
    a1 = jnp.tanh(
        jnp.dot(h_ref[...], w1_ref[...], preferred_element_type=jnp.float32)
        + b1_ref[...]
    )
    logits = (
        jnp.dot(a1, w2_ref[...], preferred_element_type=jnp.float32) + b2_ref[...]
    )

    m1 = jnp.max(logits, axis=-1, keepdims=True)
    is_max = logits == m1
    m2 = jnp.max(jnp.where(is_max, -jnp.inf, logits), axis=-1, keepdims=True)

    use_dense = flag_ref[0, 0] != 0

    @pl.when(jnp.logical_not(use_dense))
    def _sparse():
        t = jnp.exp(m2 - m1)
        p2 = t / (1.0 + t)
        p1 = 1.0 - p2
        out_ref[...] = jnp.where(
            is_max, p1, jnp.where(logits >= m2, p2, 0.0)
        )

    @pl.when(use_dense)
    def _dense():
        e = jnp.exp(logits - m1)
        out_ref[...] = e / jnp.sum(e, axis=-1, keepdims=True)


@jax.jit
def _gating(h, W1, b1, W2, b2, flag):
    tokens = h.shape[0]
    grid = (tokens // _BT,)
    return pl.pallas_call(
        _gating_body,
        grid=grid,
        in_specs=[
            pl.BlockSpec((1, 1), lambda i: (0, 0)),
            pl.BlockSpec((_BT, _HIDDEN), lambda i: (i, 0)),
            pl.BlockSpec((_HIDDEN, _HIDDEN), lambda i: (0, 0)),
            pl.BlockSpec((1, _HIDDEN), lambda i: (0, 0)),
            pl.BlockSpec((_HIDDEN, _EXPERTS), lambda i: (0, 0)),
            pl.BlockSpec((1, _EXPERTS), lambda i: (0, 0)),
        ],
        out_specs=pl.BlockSpec((_BT, _EXPERTS), lambda i: (i, 0)),
        out_shape=jax.ShapeDtypeStruct((tokens, _EXPERTS), jnp.float32),
    )(flag, h, W1.reshape(_HIDDEN, _HIDDEN), b1.reshape(1, _HIDDEN),
      W2.reshape(_HIDDEN, _EXPERTS), b2.reshape(1, _EXPERTS))


def kernel(h, W1, b1, W2, b2, epoch, top_k):
    warmup_epochs = 0
    if epoch is None or top_k is None:
        flag = jnp.ones((1, 1), jnp.float32)
    else:
        use_dense = (epoch < warmup_epochs) | (top_k <= 0)
        flag = jnp.asarray(use_dense, jnp.float32).reshape(1, 1)
    return _gating(h, W1, b1, W2, b2, flag)
